# Initial kernel scaffold; baseline (speedup 1.0000x reference)
import jax
import jax.numpy as jnp
from jax.experimental import pallas as pl


def kernel(x, e, edge_index, read_data, read_length, params):
    raise NotImplementedError("write your pallas kernel here")


# speedup vs baseline: 4.7903x; 4.7903x over previous
import functools

import numpy as np
import jax
import jax.numpy as jnp
from jax import lax
from jax.experimental import pallas as pl
from jax.experimental.pallas import tpu as pltpu
from jax.experimental.pallas import tpu_sc as plsc

N = 10000
E = 160000
H = 64

_NC, _NS = 2, 16
_CH = 80
_K = 5
_SUP = _CH * _K
_PER_TILE = E // _NS
_NIT = _PER_TILE // _SUP
_CHS = 40
_KS = 5
_SUPS = _CHS * _KS
_NITS = _PER_TILE // _SUPS
_ROWS_T = 624
_ROWS_LAST = N - 15 * _ROWS_T

_NB = 2000
_NBF = 400
_EB = 2000


def _np_repeat_mats():
    R8 = np.zeros((8, 256), np.float32)
    for i in range(8):
        R8[i, i * 32:(i + 1) * 32] = 1.0
    R32 = np.zeros((32, 256), np.float32)
    for i in range(8):
        for s in range(32):
            R32[s, i * 32 + s] = 1.0
    R16 = np.zeros((16, 512), np.float32)
    R16[:8, :256] = R8
    R16[8:, 256:] = R8
    R64 = np.zeros((64, 512), np.float32)
    R64[:32, :256] = R32
    R64[32:, 256:] = R32
    return R16, R64, R8.T.copy()


_R16_NP, _R64_NP, _R8T_NP = _np_repeat_mats()


def _np_conv_sel():
    es = []
    for k in range(4):
        ek = np.zeros((256, 128), np.float32)
        for lp in range(16):
            l = lp + 3 - k
            if 0 <= l < 16:
                for i in range(8):
                    ek[lp * 16 + i, l * 8 + i] = 1.0
        es.append(ek)
    sz = np.zeros((256, 128), np.float32)
    for l in range(16):
        for i in range(8):
            sz[l * 16 + 8 + i, l * 8 + i] = 1.0
    return es, sz


_CONV_E_NP, _SZ_NP = _np_conv_sel()


def _sig(z):
    return 1.0 / (1.0 + jnp.exp(-z))


def _silu(z):
    return z * _sig(z)


def _softplus(z):
    zc = jnp.minimum(z, 20.0)
    return jnp.where(z > 20.0, z, jnp.log(1.0 + jnp.exp(zc)))


def _dot(a, b):
    return jnp.dot(a, b, preferred_element_type=jnp.float32)


def _node_first_body(x_ref, rd_ref, idx_ref,
                     w1n, b1n, w2n, b2n,
                     wbig, cbig, cbtile, wxbig, wdt, bdt, aflat, dtile,
                     woutbig, szmat,
                     r16, r64, r8t, lbt, blb,
                     wa, wb, wc, wd,
                     a1w, a1b, a2w, a2b, a3w, a3b, bb1w, bb1b, bb2w, bb2b,
                     h_ref, a1_ref, tu_ref, tv_ref):
    x = x_ref[...]
    reads = rd_ref[...]
    idxc = idx_ref[...]
    h = jnp.maximum(_dot(x, w1n[...]) + b1n[...], 0.0)
    h = _dot(h, w2n[...]) + b2n[...]

    xz_all = _dot(reads, wbig[...])
    xc_all = _silu(_dot(xz_all, cbig[...]) + cbtile[...])
    z_all = _dot(xz_all, szmat[...])
    xdbl = _dot(xc_all, wxbig[...])
    hstate = jnp.zeros((x.shape[0], 256), jnp.float32)
    wdtv = wdt[...]
    bdtv = bdt[...]
    afl = aflat[...]
    r16v = r16[...]
    r64v = r64[...]
    r8tv = r8t[...]
    ys = []
    for l in range(16):
        xc_l = xc_all[:, 8 * l:8 * l + 8]
        dt = xdbl[:, 65 * l:65 * l + 1]
        b_l = xdbl[:, 65 * l + 1:65 * l + 33]
        c_l = xdbl[:, 65 * l + 33:65 * l + 65]
        delta = _softplus(dt * wdtv + bdtv)
        dxc = _dot(jnp.concatenate([delta, xc_l], axis=1), r16v)
        d_r, xc_r = dxc[:, :256], dxc[:, 256:]
        bc = _dot(jnp.concatenate([b_l, c_l], axis=1), r64v)
        b_r, c_r = bc[:, :256], bc[:, 256:]
        d_a = jnp.exp(d_r * afl)
        hstate = d_a * hstate + d_r * b_r * xc_r
        ys.append(_dot(hstate * c_r, r8tv))
    y_all = jnp.concatenate(ys, axis=1)
    y_all = (y_all + xc_all * dtile[...]) * _silu(z_all)
    o_all = _dot(y_all, woutbig[...])
    sel = jnp.zeros((x.shape[0], 4), jnp.float32)
    for l in range(16):
        sel = sel + jnp.where(idxc == l, o_all[:, 4 * l:4 * l + 4], 0.0)
    x2 = _dot(sel, lbt[...]) + blb[...]

    pu = _dot(h, wa[...]) + _dot(x2, wc[...])
    pv = _dot(h, wb[...]) + _dot(x2, wd[...])
    a1 = _dot(h, a1w[...]) + a1b[...]
    a2 = _dot(h, a2w[...]) + a2b[...]
    a3 = _dot(h, a3w[...]) + a3b[...]
    bm1 = _dot(h, bb1w[...]) + bb1b[...]
    bm2 = _dot(h, bb2w[...]) + bb2b[...]
    pad = jnp.zeros_like(a2)
    h_ref[...] = h
    a1_ref[...] = a1
    tu_ref[...] = jnp.concatenate([pu, bm1, a2, pad], axis=1)
    tv_ref[...] = jnp.concatenate([pv, bm2, a3, pad], axis=1)


def _edge_first_body(gu_ref, gv_ref, b1e, w2e, b2e, wb3, bb3,
                     ee_ref, pv_ref, pu_ref):
    gu = gu_ref[...]
    gv = gv_ref[...]
    he = jnp.maximum(gu[:, :64] + gv[:, :64] + b1e[...], 0.0)
    ee1 = jnp.maximum(_dot(he, w2e[...]) + b2e[...], 0.0)
    b3 = _dot(ee1, wb3[...]) + bb3[...]
    ehat = gu[:, 64:128] + gv[:, 64:128] + b3
    ee2 = jnp.maximum(ehat, 0.0) + ee1
    sg = _sig(ee2)
    ee_ref[...] = ee2
    pv_ref[...] = jnp.concatenate([sg * gu[:, 128:192], sg], axis=1)
    pu_ref[...] = jnp.concatenate([sg * gv[:, 128:192], sg], axis=1)


def _node_mid_body(h_ref, a1_ref, av_ref, au_ref,
                   a1w, a1b, a2w, a2b, a3w, a3b, bb1w, bb1b, bb2w, bb2b,
                   ho_ref, a1o_ref, tu_ref, tv_ref):
    accv = av_ref[...]
    accu = au_ref[...]
    hf = accv[:, :64] / (accv[:, 64:] + 1e-6)
    hb = accu[:, :64] / (accu[:, 64:] + 1e-6)
    h = jnp.maximum(a1_ref[...] + hf + hb, 0.0) + h_ref[...]
    a1 = _dot(h, a1w[...]) + a1b[...]
    a2 = _dot(h, a2w[...]) + a2b[...]
    a3 = _dot(h, a3w[...]) + a3b[...]
    bm1 = _dot(h, bb1w[...]) + bb1b[...]
    bm2 = _dot(h, bb2w[...]) + bb2b[...]
    ho_ref[...] = h
    a1o_ref[...] = a1
    tu_ref[...] = jnp.concatenate([bm1, a2], axis=1)
    tv_ref[...] = jnp.concatenate([bm2, a3], axis=1)


def _edge_mid_body(ee_ref, gu_ref, gv_ref, wb3, bb3,
                   eeo_ref, pv_ref, pu_ref):
    ee = ee_ref[...]
    gu = gu_ref[...]
    gv = gv_ref[...]
    b3 = _dot(ee, wb3[...]) + bb3[...]
    ehat = gu[:, :64] + gv[:, :64] + b3
    ee2 = jnp.maximum(ehat, 0.0) + ee
    sg = _sig(ee2)
    eeo_ref[...] = ee2
    pv_ref[...] = jnp.concatenate([sg * gu[:, 64:], sg], axis=1)
    pu_ref[...] = jnp.concatenate([sg * gv[:, 64:], sg], axis=1)


def _node_last_body(h_ref, a1_ref, av_ref, au_ref, qu_w, qv_w, bp1,
                    tq_ref):
    accv = av_ref[...]
    accu = au_ref[...]
    hf = accv[:, :64] / (accv[:, 64:] + 1e-6)
    hb = accu[:, :64] / (accu[:, 64:] + 1e-6)
    h = jnp.maximum(a1_ref[...] + hf + hb, 0.0) + h_ref[...]
    tq_ref[...] = jnp.concatenate([_dot(h, qu_w[...]) + bp1[...],
                                   _dot(h, qv_w[...])], axis=1)


def _pred_body(gu_ref, gv_ref, ee_ref, p1e, p2, bp2, out_ref):
    ph = jnp.maximum(gu_ref[:, :64] + gv_ref[:, 64:128]
                     + _dot(ee_ref[...], p1e[...]), 0.0)
    out_ref[...] = _dot(ph, p2[...]) + bp2[...]


def _full_spec(arr):
    nd = arr.ndim
    return pl.BlockSpec(arr.shape, lambda i, _nd=nd: (0,) * _nd)


def _row_spec(rows, cols):
    return pl.BlockSpec((rows, cols), lambda i: (i, 0))


def _tc_call(body, row_inputs, full_inputs, out_cols, total_rows, block_rows):
    grid = (total_rows // block_rows,)
    in_specs = ([_row_spec(block_rows, a.shape[1]) for a in row_inputs]
                + [_full_spec(a) for a in full_inputs])
    out_specs = [_row_spec(block_rows, w) for w in out_cols]
    out_shape = [jax.ShapeDtypeStruct((total_rows, w), jnp.float32)
                 for w in out_cols]
    f = pl.pallas_call(body, grid=grid, in_specs=in_specs,
                       out_specs=out_specs, out_shape=out_shape)
    return f(*row_inputs, *full_inputs)


def _sc_mesh():
    return plsc.VectorSubcoreMesh(core_axis_name="c", subcore_axis_name="s",
                                  num_cores=_NC, num_subcores=_NS)


@functools.lru_cache(maxsize=None)
def _make_gather(width):
    out_t = [jax.ShapeDtypeStruct((E, width), jnp.float32)] * 2

    @functools.partial(
        pl.kernel, out_type=out_t, mesh=_sc_mesh(),
        scratch_types=[pltpu.VMEM((_CH,), jnp.int32)] * _K
        + [pltpu.VMEM((_SUP, width), jnp.float32),
           pltpu.SemaphoreType.DMA])
    def gk(tab_u, tab_v, u1, v1, out_u, out_v, *scr):
        idxb = scr[:_K]
        rowb, sem = scr[_K], scr[_K + 1]
        s = lax.axis_index("s")
        c = lax.axis_index("c")

        def run(tab, idxsrc, out):
            def body(i, carry):
                base = s * _PER_TILE + i * _SUP
                for k in range(_K):
                    pltpu.sync_copy(idxsrc.at[pl.ds(base + k * _CH, _CH)],
                                    idxb[k])
                descs = [pltpu.async_copy(tab.at[idxb[k]],
                                          rowb.at[pl.ds(k * _CH, _CH)], sem)
                         for k in range(_K)]
                for d in descs:
                    d.wait()
                pltpu.sync_copy(rowb, out.at[pl.ds(base, _SUP)])
                return carry
            lax.fori_loop(0, _NIT, body, 0)

        @pl.when(c == 0)
        def _():
            run(tab_u, u1, out_u)

        @pl.when(c == 1)
        def _():
            run(tab_v, v1, out_v)

    return gk


@functools.lru_cache(maxsize=None)
def _make_scatter():
    out_t = [jax.ShapeDtypeStruct((N, 128), jnp.float32)] * 2

    @functools.partial(
        pl.kernel, out_type=out_t, mesh=_sc_mesh(),
        scratch_types=[pltpu.VMEM((_CHS,), jnp.int32)] * _KS
        + [pltpu.VMEM((_SUPS, 128), jnp.float32),
           pltpu.VMEM_SHARED((N, 128), jnp.float32)])
    def sk(pv, pu, v1, u1, zrows, out_v, out_u, *scr):
        idxb = scr[:_KS]
        datb, acc = scr[_KS], scr[_KS + 1]
        s = lax.axis_index("s")
        c = lax.axis_index("c")

        def _tilewise(src, dst):
            @pl.when(s < 15)
            def _():
                pltpu.sync_copy(src.at[pl.ds(s * _ROWS_T, _ROWS_T)],
                                dst.at[pl.ds(s * _ROWS_T, _ROWS_T)])

            @pl.when(s == 15)
            def _():
                pltpu.sync_copy(src.at[pl.ds(15 * _ROWS_T, _ROWS_LAST)],
                                dst.at[pl.ds(15 * _ROWS_T, _ROWS_LAST)])

        _tilewise(zrows, acc)
        plsc.subcore_barrier()

        def run(pay, idxsrc):
            def body(i, carry):
                base = s * _PER_TILE + i * _SUPS
                for k in range(_KS):
                    pltpu.sync_copy(idxsrc.at[pl.ds(base + k * _CHS, _CHS)],
                                    idxb[k])
                pltpu.sync_copy(pay.at[pl.ds(base, _SUPS)], datb)
                for k in range(_KS):
                    pltpu.sync_copy(datb.at[pl.ds(k * _CHS, _CHS)],
                                    acc.at[idxb[k]], add=True)
                return carry
            lax.fori_loop(0, _NITS, body, 0)

        @pl.when(c == 0)
        def _():
            run(pv, v1)

        @pl.when(c == 1)
        def _():
            run(pu, u1)

        plsc.subcore_barrier()

        @pl.when(c == 0)
        def _():
            _tilewise(acc, out_v)

        @pl.when(c == 1)
        def _():
            _tilewise(acc, out_u)

    return sk


def _sc_gather(tab_u, tab_v, u1, v1):
    return _make_gather(tab_u.shape[1])(tab_u, tab_v, u1, v1)


def _sc_scatter(pv, pu, v1, u1, zrows):
    return _make_scatter()(pv, pu, v1, u1, zrows)


def kernel(x, e, edge_index, read_data, read_length, params):
    del e
    p = params
    t = lambda q: q['W'].T
    bb = lambda q: q['b'][None, :]

    u2 = edge_index[0]
    v2 = edge_index[1]
    reads64 = read_data.reshape(N, 64)
    idxc = jnp.clip(read_length - 1, 0, 15).astype(jnp.int32)[:, None]

    mb = p['mamba']
    eye16 = jnp.eye(16, dtype=jnp.float32)
    wbig = jnp.kron(eye16, t(mb['in_proj']))
    convw = mb['conv_W'].T
    cbig = sum(jnp.asarray(_CONV_E_NP[k]) * jnp.tile(convw[k], 16)[None, :]
               for k in range(4))
    cbtile = jnp.tile(mb['conv_b'], 16)[None, :]
    wxbig = jnp.kron(eye16, t(mb['x_proj']))
    wdt = t(mb['dt_proj'])
    bdt = mb['dt_proj']['b'][None, :]
    aflat = (-jnp.exp(mb['A_log'])).reshape(1, 256)
    dtile = jnp.tile(mb['D'], 16)[None, :]
    woutbig = jnp.kron(eye16, t(mb['out_proj']))
    szmat = jnp.asarray(_SZ_NP)
    r16 = jnp.asarray(_R16_NP)
    r64 = jnp.asarray(_R64_NP)
    r8t = jnp.asarray(_R8T_NP)

    w1t = t(p['l1e'])
    wa, wb_, wc, wd = w1t[0:64], w1t[64:128], w1t[128:192], w1t[192:256]
    b1e = bb(p['l1e'])
    w2e, b2e = t(p['l2e']), bb(p['l2e'])

    g0 = p['gnn'][0]
    nf_full = [t(p['l1n']), bb(p['l1n']), t(p['l2n']), bb(p['l2n']),
               wbig, cbig, cbtile, wxbig, wdt, bdt, aflat, dtile,
               woutbig, szmat,
               r16, r64, r8t, t(p['lbase']), bb(p['lbase']),
               wa, wb_, wc, wd,
               t(g0['A1']), bb(g0['A1']), t(g0['A2']), bb(g0['A2']),
               t(g0['A3']), bb(g0['A3']), t(g0['B1']), bb(g0['B1']),
               t(g0['B2']), bb(g0['B2'])]
    h1, a1h, tu, tv = _tc_call(_node_first_body, [x, reads64, idxc], nf_full,
                               [64, 64, 256, 256], N, _NBF)

    zrows = jnp.zeros((N, 128), jnp.float32)

    gu, gv = _sc_gather(tu, tv, u2, v2)
    ee, pv, pu = _tc_call(_edge_first_body, [gu, gv],
                          [b1e, w2e, b2e, t(g0['B3']), bb(g0['B3'])],
                          [64, 128, 128], E, _EB)
    accv, accu = _sc_scatter(pv, pu, v2, u2, zrows)

    h = h1
    for li in (1, 2):
        g = p['gnn'][li]
        nm_full = [t(g['A1']), bb(g['A1']), t(g['A2']), bb(g['A2']),
                   t(g['A3']), bb(g['A3']), t(g['B1']), bb(g['B1']),
                   t(g['B2']), bb(g['B2'])]
        h, a1h, tu, tv = _tc_call(_node_mid_body, [h, a1h, accv, accu],
                                  nm_full, [64, 64, 128, 128], N, _NB)
        gu, gv = _sc_gather(tu, tv, u2, v2)
        ee, pv, pu = _tc_call(_edge_mid_body, [ee, gu, gv],
                              [t(g['B3']), bb(g['B3'])],
                              [64, 128, 128], E, _EB)
        accv, accu = _sc_scatter(pv, pu, v2, u2, zrows)

    p1t = t(p['pred1'])
    tq = _tc_call(_node_last_body, [h, a1h, accv, accu],
                  [p1t[0:64], p1t[64:128], bb(p['pred1'])], [128], N, _NB)[0]
    gqu, gqv = _sc_gather(tq, tq, u2, v2)
    scores = _tc_call(_pred_body, [gqu, gqv, ee],
                      [p1t[128:192], t(p['pred2']), bb(p['pred2'])],
                      [1], E, _EB)[0]
    return scores


# speedup vs baseline: 4.9697x; 1.0375x over previous
import functools

import numpy as np
import jax
import jax.numpy as jnp
from jax import lax
from jax.experimental import pallas as pl
from jax.experimental.pallas import tpu as pltpu
from jax.experimental.pallas import tpu_sc as plsc

N = 10000
E = 160000
H = 64

_NC, _NS = 2, 16
_CH = 80
_K = 5
_SUP = _CH * _K
_PER_TILE = E // _NS
_NIT = _PER_TILE // _SUP
_CHS = 40
_KS = 5
_SUPS = _CHS * _KS
_NITS = _PER_TILE // _SUPS
_ROWS_T = 624
_ROWS_LAST = N - 15 * _ROWS_T

_NB = 2000
_NBF = 400
_EB = 2000


def _np_repeat_mats():
    R8 = np.zeros((8, 256), np.float32)
    for i in range(8):
        R8[i, i * 32:(i + 1) * 32] = 1.0
    R32 = np.zeros((32, 256), np.float32)
    for i in range(8):
        for s in range(32):
            R32[s, i * 32 + s] = 1.0
    R16 = np.zeros((16, 512), np.float32)
    R16[:8, :256] = R8
    R16[8:, 256:] = R8
    R64 = np.zeros((64, 512), np.float32)
    R64[:32, :256] = R32
    R64[32:, 256:] = R32
    return R16, R64, R8.T.copy()


_R16_NP, _R64_NP, _R8T_NP = _np_repeat_mats()


def _np_conv_sel():
    es = []
    for k in range(4):
        ek = np.zeros((256, 128), np.float32)
        for lp in range(16):
            l = lp + 3 - k
            if 0 <= l < 16:
                for i in range(8):
                    ek[lp * 16 + i, l * 8 + i] = 1.0
        es.append(ek)
    sz = np.zeros((256, 128), np.float32)
    for l in range(16):
        for i in range(8):
            sz[l * 16 + 8 + i, l * 8 + i] = 1.0
    return es, sz


_CONV_E_NP, _SZ_NP = _np_conv_sel()


def _sig(z):
    return 1.0 / (1.0 + jnp.exp(-z))


def _silu(z):
    return z * _sig(z)


def _softplus(z):
    zc = jnp.minimum(z, 20.0)
    return jnp.where(z > 20.0, z, jnp.log(1.0 + jnp.exp(zc)))


def _dot(a, b):
    return jnp.dot(a, b, preferred_element_type=jnp.float32)


def _node_first_body(x_ref, rd_ref, idx_ref,
                     w1n, b1n, w2n, b2n,
                     wbig, cbig, cbtile, wxbig, wdt, bdt, aflat, dtile,
                     woutbig, szmat,
                     r16, r64, r8t, lbt, blb,
                     wa, wb, wc, wd,
                     a1w, a1b, a2w, a2b, a3w, a3b, bb1w, bb1b, bb2w, bb2b,
                     h_ref, a1_ref, tu_ref, tv_ref):
    x = x_ref[...]
    reads = rd_ref[...]
    idxc = idx_ref[...]
    h = jnp.maximum(_dot(x, w1n[...]) + b1n[...], 0.0)
    h = _dot(h, w2n[...]) + b2n[...]

    xz_all = _dot(reads, wbig[...])
    xc_all = _silu(_dot(xz_all, cbig[...]) + cbtile[...])
    z_all = _dot(xz_all, szmat[...])
    xdbl = _dot(xc_all, wxbig[...])
    hstate = jnp.zeros((x.shape[0], 256), jnp.float32)
    wdtv = wdt[...]
    bdtv = bdt[...]
    afl = aflat[...]
    r16v = r16[...]
    r64v = r64[...]
    r8tv = r8t[...]
    ys = []
    for l in range(16):
        xc_l = xc_all[:, 8 * l:8 * l + 8]
        dt = xdbl[:, 65 * l:65 * l + 1]
        b_l = xdbl[:, 65 * l + 1:65 * l + 33]
        c_l = xdbl[:, 65 * l + 33:65 * l + 65]
        delta = _softplus(dt * wdtv + bdtv)
        dxc = _dot(jnp.concatenate([delta, xc_l], axis=1), r16v)
        d_r, xc_r = dxc[:, :256], dxc[:, 256:]
        bc = _dot(jnp.concatenate([b_l, c_l], axis=1), r64v)
        b_r, c_r = bc[:, :256], bc[:, 256:]
        d_a = jnp.exp(d_r * afl)
        hstate = d_a * hstate + d_r * b_r * xc_r
        ys.append(_dot(hstate * c_r, r8tv))
    y_all = jnp.concatenate(ys, axis=1)
    y_all = (y_all + xc_all * dtile[...]) * _silu(z_all)
    o_all = _dot(y_all, woutbig[...])
    sel = jnp.zeros((x.shape[0], 4), jnp.float32)
    for l in range(16):
        sel = sel + jnp.where(idxc == l, o_all[:, 4 * l:4 * l + 4], 0.0)
    x2 = _dot(sel, lbt[...]) + blb[...]

    pu = _dot(h, wa[...]) + _dot(x2, wc[...])
    pv = _dot(h, wb[...]) + _dot(x2, wd[...])
    a1 = _dot(h, a1w[...]) + a1b[...]
    a2 = _dot(h, a2w[...]) + a2b[...]
    a3 = _dot(h, a3w[...]) + a3b[...]
    bm1 = _dot(h, bb1w[...]) + bb1b[...]
    bm2 = _dot(h, bb2w[...]) + bb2b[...]
    pad = jnp.zeros_like(a2)
    h_ref[...] = h
    a1_ref[...] = a1
    tu_ref[...] = jnp.concatenate([pu, bm1, a2, pad], axis=1)
    tv_ref[...] = jnp.concatenate([pv, bm2, a3, pad], axis=1)


def _edge_first_body(gu_ref, gv_ref, b1e, w2e, b2e, wb3, bb3,
                     ee_ref, pv_ref, pu_ref):
    gu = gu_ref[...]
    gv = gv_ref[...]
    he = jnp.maximum(gu[:, :64] + gv[:, :64] + b1e[...], 0.0)
    ee1 = jnp.maximum(_dot(he, w2e[...]) + b2e[...], 0.0)
    b3 = _dot(ee1, wb3[...]) + bb3[...]
    ehat = gu[:, 64:128] + gv[:, 64:128] + b3
    ee2 = jnp.maximum(ehat, 0.0) + ee1
    sg = _sig(ee2)
    ee_ref[...] = ee2
    pv_ref[...] = jnp.concatenate([sg * gu[:, 128:192], sg], axis=1)
    pu_ref[...] = jnp.concatenate([sg * gv[:, 128:192], sg], axis=1)


def _node_mid_body(h_ref, a1_ref, av_ref, au_ref,
                   a1w, a1b, a2w, a2b, a3w, a3b, bb1w, bb1b, bb2w, bb2b,
                   ho_ref, a1o_ref, tu_ref, tv_ref):
    accv = av_ref[...]
    accu = au_ref[...]
    hf = accv[:, :64] / (accv[:, 64:] + 1e-6)
    hb = accu[:, :64] / (accu[:, 64:] + 1e-6)
    h = jnp.maximum(a1_ref[...] + hf + hb, 0.0) + h_ref[...]
    a1 = _dot(h, a1w[...]) + a1b[...]
    a2 = _dot(h, a2w[...]) + a2b[...]
    a3 = _dot(h, a3w[...]) + a3b[...]
    bm1 = _dot(h, bb1w[...]) + bb1b[...]
    bm2 = _dot(h, bb2w[...]) + bb2b[...]
    ho_ref[...] = h
    a1o_ref[...] = a1
    tu_ref[...] = jnp.concatenate([bm1, a2], axis=1)
    tv_ref[...] = jnp.concatenate([bm2, a3], axis=1)


def _edge_mid_body(ee_ref, gu_ref, gv_ref, wb3, bb3,
                   eeo_ref, pv_ref, pu_ref):
    ee = ee_ref[...]
    gu = gu_ref[...]
    gv = gv_ref[...]
    b3 = _dot(ee, wb3[...]) + bb3[...]
    ehat = gu[:, :64] + gv[:, :64] + b3
    ee2 = jnp.maximum(ehat, 0.0) + ee
    sg = _sig(ee2)
    eeo_ref[...] = ee2
    pv_ref[...] = jnp.concatenate([sg * gu[:, 64:], sg], axis=1)
    pu_ref[...] = jnp.concatenate([sg * gv[:, 64:], sg], axis=1)


def _node_last_body(h_ref, a1_ref, av_ref, au_ref, qu_w, qv_w, bp1,
                    tq_ref):
    accv = av_ref[...]
    accu = au_ref[...]
    hf = accv[:, :64] / (accv[:, 64:] + 1e-6)
    hb = accu[:, :64] / (accu[:, 64:] + 1e-6)
    h = jnp.maximum(a1_ref[...] + hf + hb, 0.0) + h_ref[...]
    tq_ref[...] = jnp.concatenate([_dot(h, qu_w[...]) + bp1[...],
                                   _dot(h, qv_w[...])], axis=1)


def _pred_body(gu_ref, gv_ref, ee_ref, p1e, p2, bp2, out_ref):
    ph = jnp.maximum(gu_ref[:, :64] + gv_ref[:, 64:128]
                     + _dot(ee_ref[...], p1e[...]), 0.0)
    out_ref[...] = _dot(ph, p2[...]) + bp2[...]


def _full_spec(arr):
    nd = arr.ndim
    return pl.BlockSpec(arr.shape, lambda i, _nd=nd: (0,) * _nd)


def _row_spec(rows, cols):
    return pl.BlockSpec((rows, cols), lambda i: (i, 0))


def _tc_call(body, row_inputs, full_inputs, out_cols, total_rows, block_rows):
    grid = (total_rows // block_rows,)
    in_specs = ([_row_spec(block_rows, a.shape[1]) for a in row_inputs]
                + [_full_spec(a) for a in full_inputs])
    out_specs = [_row_spec(block_rows, w) for w in out_cols]
    out_shape = [jax.ShapeDtypeStruct((total_rows, w), jnp.float32)
                 for w in out_cols]
    f = pl.pallas_call(body, grid=grid, in_specs=in_specs,
                       out_specs=out_specs, out_shape=out_shape)
    return f(*row_inputs, *full_inputs)


def _sc_mesh():
    return plsc.VectorSubcoreMesh(core_axis_name="c", subcore_axis_name="s",
                                  num_cores=_NC, num_subcores=_NS)


@functools.lru_cache(maxsize=None)
def _make_gather(width):
    sup = _SUP if width <= 128 else _SUP // 2
    ch = sup // _K
    nit = _PER_TILE // sup
    out_t = [jax.ShapeDtypeStruct((E, width), jnp.float32)] * 2

    @functools.partial(
        pl.kernel, out_type=out_t, mesh=_sc_mesh(),
        scratch_types=[pltpu.VMEM((ch,), jnp.int32)] * (2 * _K)
        + [pltpu.VMEM((sup, width), jnp.float32)] * 2
        + [pltpu.SemaphoreType.DMA] * 3)
    def gk(tab_u, tab_v, u1, v1, out_u, out_v, *scr):
        idxb = [scr[:_K], scr[_K:2 * _K]]
        rowb = [scr[2 * _K], scr[2 * _K + 1]]
        gsem = scr[2 * _K + 2]
        ssem = [scr[2 * _K + 3], scr[2 * _K + 4]]
        s = lax.axis_index("s")
        c = lax.axis_index("c")

        def run(tab, idxsrc, out):
            def body(i, carry):
                base = s * _PER_TILE + i * sup
                for b in (0, 1):
                    @pl.when((i % 2 == b) & (i >= 2))
                    def _(b=b):
                        pltpu.make_async_copy(
                            rowb[b], out.at[pl.ds(base - 2 * sup, sup)],
                            ssem[b]).wait()

                    @pl.when(i % 2 == b)
                    def _(b=b):
                        for k in range(_K):
                            pltpu.sync_copy(
                                idxsrc.at[pl.ds(base + k * ch, ch)],
                                idxb[b][k])
                        descs = [pltpu.async_copy(
                            tab.at[idxb[b][k]],
                            rowb[b].at[pl.ds(k * ch, ch)], gsem)
                            for k in range(_K)]
                        for d in descs:
                            d.wait()
                        pltpu.async_copy(rowb[b], out.at[pl.ds(base, sup)],
                                         ssem[b])
                return carry
            lax.fori_loop(0, nit, body, 0)
            for b in (0, 1):
                last_i = nit - 1 - ((nit - 1 - b) % 2)
                pltpu.make_async_copy(
                    rowb[b],
                    out.at[pl.ds(s * _PER_TILE + last_i * sup, sup)],
                    ssem[b]).wait()

        @pl.when(c == 0)
        def _():
            run(tab_u, u1, out_u)

        @pl.when(c == 1)
        def _():
            run(tab_v, v1, out_v)

    return gk


@functools.lru_cache(maxsize=None)
def _make_scatter():
    out_t = [jax.ShapeDtypeStruct((N, 128), jnp.float32)] * 2

    @functools.partial(
        pl.kernel, out_type=out_t, mesh=_sc_mesh(),
        scratch_types=[pltpu.VMEM((_CHS,), jnp.int32)] * _KS
        + [pltpu.VMEM((_SUPS, 128), jnp.float32),
           pltpu.VMEM_SHARED((N, 128), jnp.float32)])
    def sk(pv, pu, v1, u1, zrows, out_v, out_u, *scr):
        idxb = scr[:_KS]
        datb, acc = scr[_KS], scr[_KS + 1]
        s = lax.axis_index("s")
        c = lax.axis_index("c")

        def _tilewise(src, dst):
            @pl.when(s < 15)
            def _():
                pltpu.sync_copy(src.at[pl.ds(s * _ROWS_T, _ROWS_T)],
                                dst.at[pl.ds(s * _ROWS_T, _ROWS_T)])

            @pl.when(s == 15)
            def _():
                pltpu.sync_copy(src.at[pl.ds(15 * _ROWS_T, _ROWS_LAST)],
                                dst.at[pl.ds(15 * _ROWS_T, _ROWS_LAST)])

        _tilewise(zrows, acc)
        plsc.subcore_barrier()

        def run(pay, idxsrc):
            def body(i, carry):
                base = s * _PER_TILE + i * _SUPS
                for k in range(_KS):
                    pltpu.sync_copy(idxsrc.at[pl.ds(base + k * _CHS, _CHS)],
                                    idxb[k])
                pltpu.sync_copy(pay.at[pl.ds(base, _SUPS)], datb)
                for k in range(_KS):
                    pltpu.sync_copy(datb.at[pl.ds(k * _CHS, _CHS)],
                                    acc.at[idxb[k]], add=True)
                return carry
            lax.fori_loop(0, _NITS, body, 0)

        @pl.when(c == 0)
        def _():
            run(pv, v1)

        @pl.when(c == 1)
        def _():
            run(pu, u1)

        plsc.subcore_barrier()

        @pl.when(c == 0)
        def _():
            _tilewise(acc, out_v)

        @pl.when(c == 1)
        def _():
            _tilewise(acc, out_u)

    return sk


def _sc_gather(tab_u, tab_v, u1, v1):
    return _make_gather(tab_u.shape[1])(tab_u, tab_v, u1, v1)


def _sc_scatter(pv, pu, v1, u1, zrows):
    return _make_scatter()(pv, pu, v1, u1, zrows)


def kernel(x, e, edge_index, read_data, read_length, params):
    del e
    p = params
    t = lambda q: q['W'].T
    bb = lambda q: q['b'][None, :]

    u2 = edge_index[0]
    v2 = edge_index[1]
    reads64 = read_data.reshape(N, 64)
    idxc = jnp.clip(read_length - 1, 0, 15).astype(jnp.int32)[:, None]

    mb = p['mamba']
    eye16 = jnp.eye(16, dtype=jnp.float32)
    wbig = jnp.kron(eye16, t(mb['in_proj']))
    convw = mb['conv_W'].T
    cbig = sum(jnp.asarray(_CONV_E_NP[k]) * jnp.tile(convw[k], 16)[None, :]
               for k in range(4))
    cbtile = jnp.tile(mb['conv_b'], 16)[None, :]
    wxbig = jnp.kron(eye16, t(mb['x_proj']))
    wdt = t(mb['dt_proj'])
    bdt = mb['dt_proj']['b'][None, :]
    aflat = (-jnp.exp(mb['A_log'])).reshape(1, 256)
    dtile = jnp.tile(mb['D'], 16)[None, :]
    woutbig = jnp.kron(eye16, t(mb['out_proj']))
    szmat = jnp.asarray(_SZ_NP)
    r16 = jnp.asarray(_R16_NP)
    r64 = jnp.asarray(_R64_NP)
    r8t = jnp.asarray(_R8T_NP)

    w1t = t(p['l1e'])
    wa, wb_, wc, wd = w1t[0:64], w1t[64:128], w1t[128:192], w1t[192:256]
    b1e = bb(p['l1e'])
    w2e, b2e = t(p['l2e']), bb(p['l2e'])

    g0 = p['gnn'][0]
    nf_full = [t(p['l1n']), bb(p['l1n']), t(p['l2n']), bb(p['l2n']),
               wbig, cbig, cbtile, wxbig, wdt, bdt, aflat, dtile,
               woutbig, szmat,
               r16, r64, r8t, t(p['lbase']), bb(p['lbase']),
               wa, wb_, wc, wd,
               t(g0['A1']), bb(g0['A1']), t(g0['A2']), bb(g0['A2']),
               t(g0['A3']), bb(g0['A3']), t(g0['B1']), bb(g0['B1']),
               t(g0['B2']), bb(g0['B2'])]
    h1, a1h, tu, tv = _tc_call(_node_first_body, [x, reads64, idxc], nf_full,
                               [64, 64, 256, 256], N, _NBF)

    zrows = jnp.zeros((N, 128), jnp.float32)

    gu, gv = _sc_gather(tu, tv, u2, v2)
    ee, pv, pu = _tc_call(_edge_first_body, [gu, gv],
                          [b1e, w2e, b2e, t(g0['B3']), bb(g0['B3'])],
                          [64, 128, 128], E, _EB)
    accv, accu = _sc_scatter(pv, pu, v2, u2, zrows)

    h = h1
    for li in (1, 2):
        g = p['gnn'][li]
        nm_full = [t(g['A1']), bb(g['A1']), t(g['A2']), bb(g['A2']),
                   t(g['A3']), bb(g['A3']), t(g['B1']), bb(g['B1']),
                   t(g['B2']), bb(g['B2'])]
        h, a1h, tu, tv = _tc_call(_node_mid_body, [h, a1h, accv, accu],
                                  nm_full, [64, 64, 128, 128], N, _NB)
        gu, gv = _sc_gather(tu, tv, u2, v2)
        ee, pv, pu = _tc_call(_edge_mid_body, [ee, gu, gv],
                              [t(g['B3']), bb(g['B3'])],
                              [64, 128, 128], E, _EB)
        accv, accu = _sc_scatter(pv, pu, v2, u2, zrows)

    p1t = t(p['pred1'])
    tq = _tc_call(_node_last_body, [h, a1h, accv, accu],
                  [p1t[0:64], p1t[64:128], bb(p['pred1'])], [128], N, _NB)[0]
    gqu, gqv = _sc_gather(tq, tq, u2, v2)
    scores = _tc_call(_pred_body, [gqu, gqv, ee],
                      [p1t[128:192], t(p['pred2']), bb(p['pred2'])],
                      [1], E, _EB)[0]
    return scores


# speedup vs baseline: 5.4589x; 1.0984x over previous
import functools

import numpy as np
import jax
import jax.numpy as jnp
from jax import lax
from jax.experimental import pallas as pl
from jax.experimental.pallas import tpu as pltpu
from jax.experimental.pallas import tpu_sc as plsc

N = 10000
E = 160000
H = 64

_NC, _NS = 2, 16
_CH = 80
_K = 5
_SUP = _CH * _K
_PER_TILE = E // _NS
_NIT = _PER_TILE // _SUP
_CHS = 40
_KS = 2
_SUPS = _CHS * _KS
_NITS = _PER_TILE // _SUPS
_ROWS_T = 624
_ROWS_LAST = N - 15 * _ROWS_T

_NB = 2000
_NBF = 400
_EB = 2000


def _np_repeat_mats():
    R8 = np.zeros((8, 256), np.float32)
    for i in range(8):
        R8[i, i * 32:(i + 1) * 32] = 1.0
    R32 = np.zeros((32, 256), np.float32)
    for i in range(8):
        for s in range(32):
            R32[s, i * 32 + s] = 1.0
    R16 = np.zeros((16, 512), np.float32)
    R16[:8, :256] = R8
    R16[8:, 256:] = R8
    R64 = np.zeros((64, 512), np.float32)
    R64[:32, :256] = R32
    R64[32:, 256:] = R32
    return R16, R64, R8.T.copy()


_R16_NP, _R64_NP, _R8T_NP = _np_repeat_mats()


def _np_conv_sel():
    es = []
    for k in range(4):
        ek = np.zeros((256, 128), np.float32)
        for lp in range(16):
            l = lp + 3 - k
            if 0 <= l < 16:
                for i in range(8):
                    ek[lp * 16 + i, l * 8 + i] = 1.0
        es.append(ek)
    sz = np.zeros((256, 128), np.float32)
    for l in range(16):
        for i in range(8):
            sz[l * 16 + 8 + i, l * 8 + i] = 1.0
    return es, sz


_CONV_E_NP, _SZ_NP = _np_conv_sel()


def _sig(z):
    return 1.0 / (1.0 + jnp.exp(-z))


def _silu(z):
    return z * _sig(z)


def _softplus(z):
    zc = jnp.minimum(z, 20.0)
    return jnp.where(z > 20.0, z, jnp.log(1.0 + jnp.exp(zc)))


def _dot(a, b):
    return jnp.dot(a, b, preferred_element_type=jnp.float32)


def _node_first_body(x_ref, rd_ref, idx_ref,
                     w1n, b1n, w2n, b2n,
                     wbig, cbig, cbtile, wxbig, wdt, bdt, aflat, dtile,
                     woutbig, szmat,
                     r16, r64, r8t, lbt, blb,
                     wa, wb, wc, wd,
                     a1w, a1b, a2w, a2b, a3w, a3b, bb1w, bb1b, bb2w, bb2b,
                     h_ref, a1_ref, tu_ref, tv_ref):
    x = x_ref[...]
    reads = rd_ref[...]
    idxc = idx_ref[...]
    h = jnp.maximum(_dot(x, w1n[...]) + b1n[...], 0.0)
    h = _dot(h, w2n[...]) + b2n[...]

    xz_all = _dot(reads, wbig[...])
    xc_all = _silu(_dot(xz_all, cbig[...]) + cbtile[...])
    z_all = _dot(xz_all, szmat[...])
    xdbl = _dot(xc_all, wxbig[...])
    hstate = jnp.zeros((x.shape[0], 256), jnp.float32)
    wdtv = wdt[...]
    bdtv = bdt[...]
    afl = aflat[...]
    r16v = r16[...]
    r64v = r64[...]
    r8tv = r8t[...]
    ys = []
    for l in range(16):
        xc_l = xc_all[:, 8 * l:8 * l + 8]
        dt = xdbl[:, 65 * l:65 * l + 1]
        b_l = xdbl[:, 65 * l + 1:65 * l + 33]
        c_l = xdbl[:, 65 * l + 33:65 * l + 65]
        delta = _softplus(dt * wdtv + bdtv)
        dxc = _dot(jnp.concatenate([delta, xc_l], axis=1), r16v)
        d_r, xc_r = dxc[:, :256], dxc[:, 256:]
        bc = _dot(jnp.concatenate([b_l, c_l], axis=1), r64v)
        b_r, c_r = bc[:, :256], bc[:, 256:]
        d_a = jnp.exp(d_r * afl)
        hstate = d_a * hstate + d_r * b_r * xc_r
        ys.append(_dot(hstate * c_r, r8tv))
    y_all = jnp.concatenate(ys, axis=1)
    y_all = (y_all + xc_all * dtile[...]) * _silu(z_all)
    o_all = _dot(y_all, woutbig[...])
    sel = jnp.zeros((x.shape[0], 4), jnp.float32)
    for l in range(16):
        sel = sel + jnp.where(idxc == l, o_all[:, 4 * l:4 * l + 4], 0.0)
    x2 = _dot(sel, lbt[...]) + blb[...]

    pu = _dot(h, wa[...]) + _dot(x2, wc[...])
    pv = _dot(h, wb[...]) + _dot(x2, wd[...])
    a1 = _dot(h, a1w[...]) + a1b[...]
    a2 = _dot(h, a2w[...]) + a2b[...]
    a3 = _dot(h, a3w[...]) + a3b[...]
    bm1 = _dot(h, bb1w[...]) + bb1b[...]
    bm2 = _dot(h, bb2w[...]) + bb2b[...]
    pad = jnp.zeros_like(a2)
    h_ref[...] = h
    a1_ref[...] = a1
    tu_ref[...] = jnp.concatenate([pu, bm1, a2, pad], axis=1)
    tv_ref[...] = jnp.concatenate([pv, bm2, a3, pad], axis=1)


def _edge_first_body(gu_ref, gv_ref, b1e, w2e, b2e, wb3, bb3,
                     ee_ref, pv_ref, pu_ref):
    gu = gu_ref[...]
    gv = gv_ref[...]
    he = jnp.maximum(gu[:, :64] + gv[:, :64] + b1e[...], 0.0)
    ee1 = jnp.maximum(_dot(he, w2e[...]) + b2e[...], 0.0)
    b3 = _dot(ee1, wb3[...]) + bb3[...]
    ehat = gu[:, 64:128] + gv[:, 64:128] + b3
    ee2 = jnp.maximum(ehat, 0.0) + ee1
    sg = _sig(ee2)
    ee_ref[...] = ee2
    pv_ref[...] = jnp.concatenate([sg * gu[:, 128:192], sg], axis=1)
    pu_ref[...] = jnp.concatenate([sg * gv[:, 128:192], sg], axis=1)


def _node_mid_body(h_ref, a1_ref, av_ref, au_ref,
                   a1w, a1b, a2w, a2b, a3w, a3b, bb1w, bb1b, bb2w, bb2b,
                   ho_ref, a1o_ref, tu_ref, tv_ref):
    accv = av_ref[...]
    accu = au_ref[...]
    hf = accv[:, :64] / (accv[:, 64:] + 1e-6)
    hb = accu[:, :64] / (accu[:, 64:] + 1e-6)
    h = jnp.maximum(a1_ref[...] + hf + hb, 0.0) + h_ref[...]
    a1 = _dot(h, a1w[...]) + a1b[...]
    a2 = _dot(h, a2w[...]) + a2b[...]
    a3 = _dot(h, a3w[...]) + a3b[...]
    bm1 = _dot(h, bb1w[...]) + bb1b[...]
    bm2 = _dot(h, bb2w[...]) + bb2b[...]
    ho_ref[...] = h
    a1o_ref[...] = a1
    tu_ref[...] = jnp.concatenate([bm1, a2], axis=1)
    tv_ref[...] = jnp.concatenate([bm2, a3], axis=1)


def _edge_mid_body(ee_ref, gu_ref, gv_ref, wb3, bb3,
                   eeo_ref, pv_ref, pu_ref):
    ee = ee_ref[...]
    gu = gu_ref[...]
    gv = gv_ref[...]
    b3 = _dot(ee, wb3[...]) + bb3[...]
    ehat = gu[:, :64] + gv[:, :64] + b3
    ee2 = jnp.maximum(ehat, 0.0) + ee
    sg = _sig(ee2)
    eeo_ref[...] = ee2
    pv_ref[...] = jnp.concatenate([sg * gu[:, 64:], sg], axis=1)
    pu_ref[...] = jnp.concatenate([sg * gv[:, 64:], sg], axis=1)


def _node_last_body(h_ref, a1_ref, av_ref, au_ref, qu_w, qv_w, bp1,
                    tq_ref):
    accv = av_ref[...]
    accu = au_ref[...]
    hf = accv[:, :64] / (accv[:, 64:] + 1e-6)
    hb = accu[:, :64] / (accu[:, 64:] + 1e-6)
    h = jnp.maximum(a1_ref[...] + hf + hb, 0.0) + h_ref[...]
    tq_ref[...] = jnp.concatenate([_dot(h, qu_w[...]) + bp1[...],
                                   _dot(h, qv_w[...])], axis=1)


def _pred_body(gu_ref, gv_ref, ee_ref, p1e, p2, bp2, out_ref):
    ph = jnp.maximum(gu_ref[:, :64] + gv_ref[:, 64:128]
                     + _dot(ee_ref[...], p1e[...]), 0.0)
    out_ref[...] = _dot(ph, p2[...]) + bp2[...]


def _full_spec(arr):
    nd = arr.ndim
    return pl.BlockSpec(arr.shape, lambda i, _nd=nd: (0,) * _nd)


def _row_spec(rows, cols):
    return pl.BlockSpec((rows, cols), lambda i: (i, 0))


def _tc_call(body, row_inputs, full_inputs, out_cols, total_rows, block_rows):
    grid = (total_rows // block_rows,)
    in_specs = ([_row_spec(block_rows, a.shape[1]) for a in row_inputs]
                + [_full_spec(a) for a in full_inputs])
    out_specs = [_row_spec(block_rows, w) for w in out_cols]
    out_shape = [jax.ShapeDtypeStruct((total_rows, w), jnp.float32)
                 for w in out_cols]
    f = pl.pallas_call(body, grid=grid, in_specs=in_specs,
                       out_specs=out_specs, out_shape=out_shape)
    return f(*row_inputs, *full_inputs)


def _sc_mesh():
    return plsc.VectorSubcoreMesh(core_axis_name="c", subcore_axis_name="s",
                                  num_cores=_NC, num_subcores=_NS)


@functools.lru_cache(maxsize=None)
def _make_gather(width):
    sup = _SUP if width <= 128 else _SUP // 2
    ch = sup // _K
    nit = _PER_TILE // sup
    out_t = [jax.ShapeDtypeStruct((E, width), jnp.float32)] * 2

    @functools.partial(
        pl.kernel, out_type=out_t, mesh=_sc_mesh(),
        scratch_types=[pltpu.VMEM((ch,), jnp.int32)] * (2 * _K)
        + [pltpu.VMEM((sup, width), jnp.float32)] * 2
        + [pltpu.SemaphoreType.DMA] * 3)
    def gk(tab_u, tab_v, u1, v1, out_u, out_v, *scr):
        idxb = [scr[:_K], scr[_K:2 * _K]]
        rowb = [scr[2 * _K], scr[2 * _K + 1]]
        gsem = scr[2 * _K + 2]
        ssem = [scr[2 * _K + 3], scr[2 * _K + 4]]
        s = lax.axis_index("s")
        c = lax.axis_index("c")

        def run(tab, idxsrc, out):
            def body(i, carry):
                base = s * _PER_TILE + i * sup
                for b in (0, 1):
                    @pl.when((i % 2 == b) & (i >= 2))
                    def _(b=b):
                        pltpu.make_async_copy(
                            rowb[b], out.at[pl.ds(base - 2 * sup, sup)],
                            ssem[b]).wait()

                    @pl.when(i % 2 == b)
                    def _(b=b):
                        for k in range(_K):
                            pltpu.sync_copy(
                                idxsrc.at[pl.ds(base + k * ch, ch)],
                                idxb[b][k])
                        descs = [pltpu.async_copy(
                            tab.at[idxb[b][k]],
                            rowb[b].at[pl.ds(k * ch, ch)], gsem)
                            for k in range(_K)]
                        for d in descs:
                            d.wait()
                        pltpu.async_copy(rowb[b], out.at[pl.ds(base, sup)],
                                         ssem[b])
                return carry
            lax.fori_loop(0, nit, body, 0)
            for b in (0, 1):
                last_i = nit - 1 - ((nit - 1 - b) % 2)
                pltpu.make_async_copy(
                    rowb[b],
                    out.at[pl.ds(s * _PER_TILE + last_i * sup, sup)],
                    ssem[b]).wait()

        @pl.when(c == 0)
        def _():
            run(tab_u, u1, out_u)

        @pl.when(c == 1)
        def _():
            run(tab_v, v1, out_v)

    return gk


@functools.lru_cache(maxsize=None)
def _make_scatter():
    out_t = [jax.ShapeDtypeStruct((N, 128), jnp.float32)] * 2

    @functools.partial(
        pl.kernel, out_type=out_t, mesh=_sc_mesh(),
        scratch_types=[pltpu.VMEM((_CHS,), jnp.int32)] * _KS
        + [pltpu.VMEM((_SUPS, 128), jnp.float32)] * 2
        + [pltpu.VMEM_SHARED((N, 128), jnp.float32),
           pltpu.SemaphoreType.DMA, pltpu.SemaphoreType.DMA])
    def sk(pv, pu, v1, u1, zrows, out_v, out_u, *scr):
        idxb = scr[:_KS]
        datb = [scr[_KS], scr[_KS + 1]]
        acc = scr[_KS + 2]
        psem = [scr[_KS + 3], scr[_KS + 4]]
        s = lax.axis_index("s")
        c = lax.axis_index("c")

        def _tilewise(src, dst):
            @pl.when(s < 15)
            def _():
                pltpu.sync_copy(src.at[pl.ds(s * _ROWS_T, _ROWS_T)],
                                dst.at[pl.ds(s * _ROWS_T, _ROWS_T)])

            @pl.when(s == 15)
            def _():
                pltpu.sync_copy(src.at[pl.ds(15 * _ROWS_T, _ROWS_LAST)],
                                dst.at[pl.ds(15 * _ROWS_T, _ROWS_LAST)])

        _tilewise(zrows, acc)
        plsc.subcore_barrier()

        def run(pay, idxsrc):
            pltpu.async_copy(pay.at[pl.ds(s * _PER_TILE, _SUPS)],
                             datb[0], psem[0])

            def body(i, carry):
                base = s * _PER_TILE + i * _SUPS
                for b in (0, 1):
                    @pl.when(i % 2 == b)
                    def _(b=b):
                        pltpu.make_async_copy(pay.at[pl.ds(base, _SUPS)],
                                              datb[b], psem[b]).wait()

                        @pl.when(i + 1 < _NITS)
                        def _():
                            pltpu.async_copy(
                                pay.at[pl.ds(base + _SUPS, _SUPS)],
                                datb[1 - b], psem[1 - b])
                        for k in range(_KS):
                            pltpu.sync_copy(
                                idxsrc.at[pl.ds(base + k * _CHS, _CHS)],
                                idxb[k])
                            pltpu.sync_copy(datb[b].at[pl.ds(k * _CHS, _CHS)],
                                            acc.at[idxb[k]], add=True)
                return carry
            lax.fori_loop(0, _NITS, body, 0)

        @pl.when(c == 0)
        def _():
            run(pv, v1)

        @pl.when(c == 1)
        def _():
            run(pu, u1)

        plsc.subcore_barrier()

        @pl.when(c == 0)
        def _():
            _tilewise(acc, out_v)

        @pl.when(c == 1)
        def _():
            _tilewise(acc, out_u)

    return sk


def _sc_gather(tab_u, tab_v, u1, v1):
    return _make_gather(tab_u.shape[1])(tab_u, tab_v, u1, v1)


def _sc_scatter(pv, pu, v1, u1, zrows):
    return _make_scatter()(pv, pu, v1, u1, zrows)


def kernel(x, e, edge_index, read_data, read_length, params):
    del e
    p = params
    t = lambda q: q['W'].T
    bb = lambda q: q['b'][None, :]

    u2 = edge_index[0]
    v2 = edge_index[1]
    reads64 = read_data.reshape(N, 64)
    idxc = jnp.clip(read_length - 1, 0, 15).astype(jnp.int32)[:, None]

    mb = p['mamba']
    eye16 = jnp.eye(16, dtype=jnp.float32)
    wbig = jnp.kron(eye16, t(mb['in_proj']))
    convw = mb['conv_W'].T
    cbig = sum(jnp.asarray(_CONV_E_NP[k]) * jnp.tile(convw[k], 16)[None, :]
               for k in range(4))
    cbtile = jnp.tile(mb['conv_b'], 16)[None, :]
    wxbig = jnp.kron(eye16, t(mb['x_proj']))
    wdt = t(mb['dt_proj'])
    bdt = mb['dt_proj']['b'][None, :]
    aflat = (-jnp.exp(mb['A_log'])).reshape(1, 256)
    dtile = jnp.tile(mb['D'], 16)[None, :]
    woutbig = jnp.kron(eye16, t(mb['out_proj']))
    szmat = jnp.asarray(_SZ_NP)
    r16 = jnp.asarray(_R16_NP)
    r64 = jnp.asarray(_R64_NP)
    r8t = jnp.asarray(_R8T_NP)

    w1t = t(p['l1e'])
    wa, wb_, wc, wd = w1t[0:64], w1t[64:128], w1t[128:192], w1t[192:256]
    b1e = bb(p['l1e'])
    w2e, b2e = t(p['l2e']), bb(p['l2e'])

    g0 = p['gnn'][0]
    nf_full = [t(p['l1n']), bb(p['l1n']), t(p['l2n']), bb(p['l2n']),
               wbig, cbig, cbtile, wxbig, wdt, bdt, aflat, dtile,
               woutbig, szmat,
               r16, r64, r8t, t(p['lbase']), bb(p['lbase']),
               wa, wb_, wc, wd,
               t(g0['A1']), bb(g0['A1']), t(g0['A2']), bb(g0['A2']),
               t(g0['A3']), bb(g0['A3']), t(g0['B1']), bb(g0['B1']),
               t(g0['B2']), bb(g0['B2'])]
    h1, a1h, tu, tv = _tc_call(_node_first_body, [x, reads64, idxc], nf_full,
                               [64, 64, 256, 256], N, _NBF)

    zrows = jnp.zeros((N, 128), jnp.float32)

    gu, gv = _sc_gather(tu, tv, u2, v2)
    ee, pv, pu = _tc_call(_edge_first_body, [gu, gv],
                          [b1e, w2e, b2e, t(g0['B3']), bb(g0['B3'])],
                          [64, 128, 128], E, _EB)
    accv, accu = _sc_scatter(pv, pu, v2, u2, zrows)

    h = h1
    for li in (1, 2):
        g = p['gnn'][li]
        nm_full = [t(g['A1']), bb(g['A1']), t(g['A2']), bb(g['A2']),
                   t(g['A3']), bb(g['A3']), t(g['B1']), bb(g['B1']),
                   t(g['B2']), bb(g['B2'])]
        h, a1h, tu, tv = _tc_call(_node_mid_body, [h, a1h, accv, accu],
                                  nm_full, [64, 64, 128, 128], N, _NB)
        gu, gv = _sc_gather(tu, tv, u2, v2)
        ee, pv, pu = _tc_call(_edge_mid_body, [ee, gu, gv],
                              [t(g['B3']), bb(g['B3'])],
                              [64, 128, 128], E, _EB)
        accv, accu = _sc_scatter(pv, pu, v2, u2, zrows)

    p1t = t(p['pred1'])
    tq = _tc_call(_node_last_body, [h, a1h, accv, accu],
                  [p1t[0:64], p1t[64:128], bb(p['pred1'])], [128], N, _NB)[0]
    gqu, gqv = _sc_gather(tq, tq, u2, v2)
    scores = _tc_call(_pred_body, [gqu, gqv, ee],
                      [p1t[128:192], t(p['pred2']), bb(p['pred2'])],
                      [1], E, _EB)[0]
    return scores
